# Initial kernel scaffold; baseline (speedup 1.0000x reference)
import functools

import jax
import jax.numpy as jnp
from jax import lax
from jax.experimental import pallas as pl
from jax.experimental.pallas import tpu as pltpu
from jax.experimental.pallas import tpu_sc as plsc

N = 10000
E = 320000
D = 128
H = 128
NL = 3

NC = 2
NS = 16
NW = NC * NS
CHUNK = 128
NCHUNK = 79
EDGES_PER_W = CHUNK * NCHUNK
E_PAD = NW * EDGES_PER_W
NPAD = 10240
ROWS_PER_TILE = NPAD // NS

_mesh = plsc.VectorSubcoreMesh(core_axis_name="c", subcore_axis_name="s")


def _deg_body(dst_hbm, zeros_hbm, iota_hbm, out_hbm, dst_v, hist_v, iota_v,
              shared_deg):
    cid = lax.axis_index("c")
    sid = lax.axis_index("s")
    wid = cid * NS + sid
    pltpu.sync_copy(dst_hbm.at[wid], dst_v)
    pltpu.sync_copy(zeros_hbm.at[pl.ds(0, 80)], hist_v)
    pltpu.sync_copy(iota_hbm, iota_v)

    @pl.when(sid == 0)
    def _():
        pltpu.sync_copy(zeros_hbm.at[pl.ds(0, 80)], shared_deg)

    plsc.subcore_barrier()

    ones = jnp.ones((16,), jnp.float32)

    @pl.loop(0, NCHUNK)
    def _(c):
        for j in range(CHUNK // 16):
            d = dst_v[c, pl.ds(j * 16, 16)]
            plsc.addupdate_scatter(hist_v, [d >> 7, d & 127], ones)

    pltpu.sync_copy(hist_v, shared_deg.at[iota_v], add=True)
    plsc.subcore_barrier()

    @pl.when(sid == 0)
    def _():
        pltpu.sync_copy(shared_deg, out_hbm.at[cid])


_deg_kernel = functools.partial(
    pl.kernel,
    out_type=jax.ShapeDtypeStruct((NC, 80, 128), jnp.float32),
    mesh=_mesh,
    scratch_types=[
        pltpu.VMEM((NCHUNK, CHUNK), jnp.int32),
        pltpu.VMEM((80, 128), jnp.float32),
        pltpu.VMEM((80,), jnp.int32),
        pltpu.VMEM_SHARED((80, 128), jnp.float32),
    ],
)(_deg_body)


def _agg_body(h_hbm, src_hbm, dst_hbm, zeros_hbm, out_hbm, src_v, dst_v,
              rows_v, sem, shared_agg):
    cid = lax.axis_index("c")
    sid = lax.axis_index("s")
    wid = cid * NS + sid
    pltpu.sync_copy(src_hbm.at[wid], src_v)
    pltpu.sync_copy(dst_hbm.at[wid], dst_v)
    pltpu.sync_copy(zeros_hbm,
                    shared_agg.at[pl.ds(sid * ROWS_PER_TILE, ROWS_PER_TILE)])
    plsc.subcore_barrier()

    @pl.loop(0, NCHUNK)
    def _(c):
        pltpu.async_copy(h_hbm.at[src_v.at[c]], rows_v, sem).wait()
        pltpu.sync_copy(rows_v, shared_agg.at[dst_v.at[c]], add=True)

    plsc.subcore_barrier()
    pltpu.sync_copy(
        shared_agg.at[pl.ds(sid * ROWS_PER_TILE, ROWS_PER_TILE)],
        out_hbm.at[cid].at[pl.ds(sid * ROWS_PER_TILE, ROWS_PER_TILE)])


_agg_kernel = functools.partial(
    pl.kernel,
    out_type=jax.ShapeDtypeStruct((NC, NPAD, H), jnp.float32),
    mesh=_mesh,
    scratch_types=[
        pltpu.VMEM((NCHUNK, CHUNK), jnp.int32),
        pltpu.VMEM((NCHUNK, CHUNK), jnp.int32),
        pltpu.VMEM((CHUNK, H), jnp.float32),
        pltpu.SemaphoreType.DMA,
        pltpu.VMEM_SHARED((NPAD, H), jnp.float32),
    ],
)(_agg_body)


def _bn(h, gamma, beta, eps):
    mu = jnp.mean(h, axis=0, keepdims=True)
    var = jnp.mean((h - mu) * (h - mu), axis=0, keepdims=True)
    return (h - mu) * lax.rsqrt(var + eps) * gamma + beta


def _dot(a, b):
    return jnp.dot(a, b, preferred_element_type=jnp.float32)


def _dense01_body(h_ref, p0_ref, p1_ref, d0_ref, d1_ref, ws_ref, wn_ref,
                  b_ref, g_ref, be_ref, out_ref):
    deg = jnp.maximum(d0_ref[...] + d1_ref[...], 1.0)
    agg = (p0_ref[...] + p1_ref[...]) / deg
    z = _dot(h_ref[...], ws_ref[...]) + _dot(agg, wn_ref[...]) + b_ref[...]
    r = jnp.maximum(z, 0.0)
    hc = r - jnp.mean(r, axis=0, keepdims=True)
    rms = jnp.sqrt(1e-5 + jnp.sum(hc * hc) / N)
    hp = hc / rms
    out_ref[...] = _bn(hp, g_ref[...], be_ref[...], 1e-5)


_dense01 = pl.pallas_call(
    _dense01_body,
    out_shape=jax.ShapeDtypeStruct((N, H), jnp.float32),
)


def _final_body(h_ref, p0_ref, p1_ref, d0_ref, d1_ref, ws_ref, wn_ref, b_ref,
                g2_ref, be2_ref, fw1_ref, fb1_ref, fg1_ref, fbe1_ref,
                fw2_ref, fb2_ref, fg2_ref, fbe2_ref, lw_ref, lb_ref, out_ref):
    deg = jnp.maximum(d0_ref[...] + d1_ref[...], 1.0)
    agg = (p0_ref[...] + p1_ref[...]) / deg
    z = _dot(h_ref[...], ws_ref[...]) + _dot(agg, wn_ref[...]) + b_ref[...]
    h2 = _bn(z, g2_ref[...], be2_ref[...], 1e-5)
    t = _dot(h2, fw1_ref[...]) + fb1_ref[...]
    t = jnp.maximum(_bn(t, fg1_ref[...], fbe1_ref[...], 1e-3), 0.0)
    t = _dot(t, fw2_ref[...]) + fb2_ref[...]
    t = jnp.maximum(_bn(t, fg2_ref[...], fbe2_ref[...], 1e-3), 0.0)
    out_ref[...] = _dot(t, lw_ref[...]) + lb_ref[...]


_final = pl.pallas_call(
    _final_body,
    out_shape=jax.ShapeDtypeStruct((N, H), jnp.float32),
)


def kernel(x, edge_index, params):
    src = edge_index[0].astype(jnp.int32)
    dst = edge_index[1].astype(jnp.int32)
    pad = E_PAD - E
    src_p = jnp.concatenate(
        [src, jnp.zeros((pad,), jnp.int32)]).reshape(NW, NCHUNK, CHUNK)
    dst_p = jnp.concatenate(
        [dst, jnp.full((pad,), N, jnp.int32)]).reshape(NW, NCHUNK, CHUNK)
    zeros_blk = jnp.zeros((ROWS_PER_TILE, H), jnp.float32)
    iota80 = jnp.arange(80, dtype=jnp.int32)

    degp = _deg_kernel(dst_p, zeros_blk, iota80)
    d0 = degp[0].reshape(NPAD, 1)[:N]
    d1 = degp[1].reshape(NPAD, 1)[:N]

    def row(v):
        return v.reshape(1, H)

    h = x
    for i in range(NL):
        p = params['sage'][i]
        bn = params['bn'][i]
        aggp = _agg_kernel(h, src_p, dst_p, zeros_blk)
        p0 = aggp[0, :N]
        p1 = aggp[1, :N]
        if i < NL - 1:
            h = _dense01(h, p0, p1, d0, d1, p['W_self'], p['W_neigh'],
                         row(p['b']), row(bn['gamma']), row(bn['beta']))
        else:
            fc1, fc2 = params['fc']
            lat = params['latent']
            h = _final(h, p0, p1, d0, d1, p['W_self'], p['W_neigh'],
                       row(p['b']), row(bn['gamma']), row(bn['beta']),
                       fc1['W'], row(fc1['b']), row(fc1['gamma']),
                       row(fc1['beta']),
                       fc2['W'], row(fc2['b']), row(fc2['gamma']),
                       row(fc2['beta']),
                       lat['W'], row(lat['b']))
    return h


# speedup vs baseline: 4.5328x; 4.5328x over previous
import functools

import jax
import jax.numpy as jnp
from jax import lax
from jax.experimental import pallas as pl
from jax.experimental.pallas import tpu as pltpu
from jax.experimental.pallas import tpu_sc as plsc

N = 10000
E = 320000
D = 128
H = 128
NL = 3

NC = 2
NS = 16
NW = NC * NS
CHUNK = 128
NCHUNK = 79
EDGES_PER_W = CHUNK * NCHUNK
E_PAD = NW * EDGES_PER_W
NPAD = 10240
ROWS_PER_TILE = NPAD // NS

@functools.cache
def _mesh():
    return plsc.VectorSubcoreMesh(core_axis_name="c", subcore_axis_name="s",
                                  num_cores=NC, num_subcores=NS)


def _deg_body(dst_hbm, zeros_hbm, ones_hbm, out_hbm, dst_v, ones_v,
              shared_deg):
    cid = lax.axis_index("c")
    sid = lax.axis_index("s")
    wid = cid * NS + sid
    pltpu.sync_copy(dst_hbm.at[wid], dst_v)
    pltpu.sync_copy(ones_hbm, ones_v)
    pltpu.sync_copy(zeros_hbm,
                    shared_deg.at[pl.ds(sid * ROWS_PER_TILE, ROWS_PER_TILE)])
    plsc.subcore_barrier()

    @pl.loop(0, NCHUNK)
    def _(c):
        pltpu.sync_copy(ones_v, shared_deg.at[dst_v.at[c]], add=True)

    plsc.subcore_barrier()
    pltpu.sync_copy(
        shared_deg.at[pl.ds(sid * ROWS_PER_TILE, ROWS_PER_TILE)],
        out_hbm.at[cid].at[pl.ds(sid * ROWS_PER_TILE, ROWS_PER_TILE)])


@functools.cache
def _deg_kernel():
    return pl.kernel(
        _deg_body,
        out_type=jax.ShapeDtypeStruct((NC, NPAD, H), jnp.float32),
        mesh=_mesh(),
        scratch_types=[
            pltpu.VMEM((NCHUNK, CHUNK), jnp.int32),
            pltpu.VMEM((CHUNK, H), jnp.float32),
            pltpu.VMEM_SHARED((NPAD, H), jnp.float32),
        ],
    )


def _agg_body(h_hbm, src_hbm, dst_hbm, zeros_hbm, out_hbm, src_v, dst_v,
              rows_v, sem, shared_agg):
    cid = lax.axis_index("c")
    sid = lax.axis_index("s")
    wid = cid * NS + sid
    pltpu.sync_copy(src_hbm.at[wid], src_v)
    pltpu.sync_copy(dst_hbm.at[wid], dst_v)
    pltpu.sync_copy(zeros_hbm,
                    shared_agg.at[pl.ds(sid * ROWS_PER_TILE, ROWS_PER_TILE)])
    plsc.subcore_barrier()

    @pl.loop(0, NCHUNK)
    def _(c):
        pltpu.async_copy(h_hbm.at[src_v.at[c]], rows_v, sem).wait()
        pltpu.sync_copy(rows_v, shared_agg.at[dst_v.at[c]], add=True)

    plsc.subcore_barrier()
    pltpu.sync_copy(
        shared_agg.at[pl.ds(sid * ROWS_PER_TILE, ROWS_PER_TILE)],
        out_hbm.at[cid].at[pl.ds(sid * ROWS_PER_TILE, ROWS_PER_TILE)])


@functools.cache
def _agg_kernel():
    return pl.kernel(
        _agg_body,
        out_type=jax.ShapeDtypeStruct((NC, NPAD, H), jnp.float32),
        mesh=_mesh(),
        scratch_types=[
            pltpu.VMEM((NCHUNK, CHUNK), jnp.int32),
            pltpu.VMEM((NCHUNK, CHUNK), jnp.int32),
            pltpu.VMEM((CHUNK, H), jnp.float32),
            pltpu.SemaphoreType.DMA,
            pltpu.VMEM_SHARED((NPAD, H), jnp.float32),
        ],
    )


def _bn(h, gamma, beta, eps):
    mu = jnp.mean(h, axis=0, keepdims=True)
    var = jnp.mean((h - mu) * (h - mu), axis=0, keepdims=True)
    return (h - mu) * lax.rsqrt(var + eps) * gamma + beta


def _dot(a, b):
    return jnp.dot(a, b, preferred_element_type=jnp.float32)


def _dense01_body(h_ref, p0_ref, p1_ref, d0_ref, d1_ref, ws_ref, wn_ref,
                  b_ref, g_ref, be_ref, out_ref):
    deg = jnp.maximum(d0_ref[...] + d1_ref[...], 1.0)
    agg = (p0_ref[...] + p1_ref[...]) / deg
    z = _dot(h_ref[...], ws_ref[...]) + _dot(agg, wn_ref[...]) + b_ref[...]
    r = jnp.maximum(z, 0.0)
    hc = r - jnp.mean(r, axis=0, keepdims=True)
    rms = jnp.sqrt(1e-5 + jnp.sum(hc * hc) / N)
    hp = hc / rms
    out_ref[...] = _bn(hp, g_ref[...], be_ref[...], 1e-5)


_dense01 = pl.pallas_call(
    _dense01_body,
    out_shape=jax.ShapeDtypeStruct((N, H), jnp.float32),
)


def _final_body(h_ref, p0_ref, p1_ref, d0_ref, d1_ref, ws_ref, wn_ref, b_ref,
                g2_ref, be2_ref, fw1_ref, fb1_ref, fg1_ref, fbe1_ref,
                fw2_ref, fb2_ref, fg2_ref, fbe2_ref, lw_ref, lb_ref, out_ref):
    deg = jnp.maximum(d0_ref[...] + d1_ref[...], 1.0)
    agg = (p0_ref[...] + p1_ref[...]) / deg
    z = _dot(h_ref[...], ws_ref[...]) + _dot(agg, wn_ref[...]) + b_ref[...]
    h2 = _bn(z, g2_ref[...], be2_ref[...], 1e-5)
    t = _dot(h2, fw1_ref[...]) + fb1_ref[...]
    t = jnp.maximum(_bn(t, fg1_ref[...], fbe1_ref[...], 1e-3), 0.0)
    t = _dot(t, fw2_ref[...]) + fb2_ref[...]
    t = jnp.maximum(_bn(t, fg2_ref[...], fbe2_ref[...], 1e-3), 0.0)
    out_ref[...] = _dot(t, lw_ref[...]) + lb_ref[...]


_final = pl.pallas_call(
    _final_body,
    out_shape=jax.ShapeDtypeStruct((N, H), jnp.float32),
)


def kernel(x, edge_index, params):
    src = edge_index[0].astype(jnp.int32)
    dst = edge_index[1].astype(jnp.int32)
    pad = E_PAD - E
    src_p = jnp.concatenate(
        [src, jnp.zeros((pad,), jnp.int32)]).reshape(NW, NCHUNK, CHUNK)
    dst_p = jnp.concatenate(
        [dst, jnp.full((pad,), N, jnp.int32)]).reshape(NW, NCHUNK, CHUNK)
    zeros_blk = jnp.zeros((ROWS_PER_TILE, H), jnp.float32)
    ones_blk = jnp.ones((CHUNK, H), jnp.float32)

    degp = _deg_kernel()(dst_p, zeros_blk, ones_blk)
    d0 = degp[0, :N, 0:1]
    d1 = degp[1, :N, 0:1]

    def row(v):
        return v.reshape(1, H)

    h = x
    for i in range(NL):
        p = params['sage'][i]
        bn = params['bn'][i]
        aggp = _agg_kernel()(h, src_p, dst_p, zeros_blk)
        p0 = aggp[0, :N]
        p1 = aggp[1, :N]
        if i < NL - 1:
            h = _dense01(h, p0, p1, d0, d1, p['W_self'], p['W_neigh'],
                         row(p['b']), row(bn['gamma']), row(bn['beta']))
        else:
            fc1, fc2 = params['fc']
            lat = params['latent']
            h = _final(h, p0, p1, d0, d1, p['W_self'], p['W_neigh'],
                       row(p['b']), row(bn['gamma']), row(bn['beta']),
                       fc1['W'], row(fc1['b']), row(fc1['gamma']),
                       row(fc1['beta']),
                       fc2['W'], row(fc2['b']), row(fc2['gamma']),
                       row(fc2['beta']),
                       lat['W'], row(lat['b']))
    return h
